# baseline (speedup 1.0000x reference)
import jax
import jax.numpy as jnp
from jax.experimental import pallas as pl
from jax.experimental.pallas import tpu as pltpu

S = 2048
DIM = 1024
V0, V1, V2 = 10000, 20000, 30000
VTOT = V0 + V1 + V2
K1, K2 = DIM // 4, DIM // 16
VT = 1024
NB = 59
LAST_W = VTOT - (NB - 1) * VT
T1_START = V0 // VT
T2_START = (V0 + V1) // VT
PAD1 = V0 - T1_START * VT
PAD2 = (V0 + V1) - T2_START * VT
NB1 = (PAD1 + V1 + VT - 1) // VT
NB2 = (PAD2 + V2 + VT - 1) // VT
DN = (((1,), (1,)), ((), ()))


def _proj_body(h_ref, p1_ref, p2_ref, cw_ref, cb_ref, hb_ref, h1_ref, h2_ref,
               lt_ref):
    h = h_ref[...]
    hb_ref[...] = h.astype(jnp.bfloat16)
    h1_ref[...] = jax.lax.dot_general(
        h, p1_ref[...], (((1,), (0,)), ((), ())),
        preferred_element_type=jnp.float32).astype(jnp.bfloat16)
    h2_ref[...] = jax.lax.dot_general(
        h, p2_ref[...], (((1,), (0,)), ((), ())),
        preferred_element_type=jnp.float32).astype(jnp.bfloat16)
    lt = jax.lax.dot_general(h, cw_ref[...], DN,
                             preferred_element_type=jnp.float32)
    lt_ref[...] = lt + cb_ref[...]


def _logits_body(h_ref, h1_ref, h2_ref, lt_ref, e0_ref, e1_ref, e2_ref,
                 out_ref, scratch, last_scratch, sems, last_sem):
    v = pl.program_id(0)
    slot = jax.lax.rem(v, 2)

    def out_copy(step, slot):
        return pltpu.make_async_copy(
            scratch.at[slot],
            out_ref.at[:, pl.ds(step * VT, VT)],
            sems.at[slot])

    last_copy = pltpu.make_async_copy(
        last_scratch,
        out_ref.at[:, pl.ds((NB - 1) * VT, LAST_W)],
        last_sem)

    @pl.when(jnp.logical_and(v >= 2, v < NB - 1))
    def _():
        out_copy(v - 2, slot).wait()

    def head_mm():
        return jax.lax.dot_general(h_ref[...], e0_ref[...], DN,
                                   preferred_element_type=jnp.float32)

    def t1_mm():
        mm = jax.lax.dot_general(h1_ref[...], e1_ref[...], DN,
                                 preferred_element_type=jnp.float32)
        return mm + lt_ref[:, 0:1]

    def t2_mm():
        mm = jax.lax.dot_general(h2_ref[...], e2_ref[...], DN,
                                 preferred_element_type=jnp.float32)
        return mm + lt_ref[:, 1:2]

    col = jax.lax.broadcasted_iota(jnp.int32, (1, VT), 1) + v * VT

    @pl.when(v < T1_START)
    def _head():
        scratch[slot] = head_mm()

    @pl.when(v == T1_START)
    def _mixed1():
        scratch[slot] = jnp.where(col < V0, head_mm(), t1_mm())

    @pl.when(jnp.logical_and(v > T1_START, v < T2_START))
    def _tail1():
        scratch[slot] = t1_mm()

    @pl.when(v == T2_START)
    def _mixed2():
        scratch[slot] = jnp.where(col < V0 + V1, t1_mm(), t2_mm())

    @pl.when(jnp.logical_and(v > T2_START, v < NB - 1))
    def _tail2():
        scratch[slot] = t2_mm()

    @pl.when(v < NB - 1)
    def _():
        out_copy(v, slot).start()

    @pl.when(v == NB - 1)
    def _last():
        last_scratch[...] = t2_mm()[:, :LAST_W]
        last_copy.start()
        out_copy(v - 2, slot).wait()
        out_copy(v - 1, 1 - slot).wait()
        last_copy.wait()


def kernel(hidden, input, embed0_w, embed1_w, embed2_w, proj1_w, proj2_w,
           classes_w, classes_b):
    del input
    h = hidden.reshape(S, DIM)
    cb = classes_b.reshape(1, 2)

    e0b = embed0_w.astype(jnp.bfloat16)
    e1p = jnp.pad(embed1_w.astype(jnp.bfloat16),
                  ((PAD1, NB1 * VT - PAD1 - V1), (0, 0)))
    e2p = jnp.pad(embed2_w.astype(jnp.bfloat16),
                  ((PAD2, NB2 * VT - PAD2 - V2), (0, 0)))

    hb, h1, h2, lt = pl.pallas_call(
        _proj_body,
        out_shape=(
            jax.ShapeDtypeStruct((S, DIM), jnp.bfloat16),
            jax.ShapeDtypeStruct((S, K1), jnp.bfloat16),
            jax.ShapeDtypeStruct((S, K2), jnp.bfloat16),
            jax.ShapeDtypeStruct((S, 2), jnp.float32),
        ),
    )(h, proj1_w, proj2_w, classes_w, cb)

    out = pl.pallas_call(
        _logits_body,
        grid=(NB,),
        in_specs=[
            pl.BlockSpec((S, DIM), lambda v: (0, 0)),
            pl.BlockSpec((S, K1), lambda v: (0, 0)),
            pl.BlockSpec((S, K2), lambda v: (0, 0)),
            pl.BlockSpec((S, 2), lambda v: (0, 0)),
            pl.BlockSpec((VT, DIM), lambda v: (jnp.minimum(v, T1_START), 0)),
            pl.BlockSpec((VT, K1),
                         lambda v: (jnp.clip(v - T1_START, 0, NB1 - 1), 0)),
            pl.BlockSpec((VT, K2),
                         lambda v: (jnp.clip(v - T2_START, 0, NB2 - 1), 0)),
        ],
        out_specs=pl.BlockSpec(memory_space=pl.ANY),
        out_shape=jax.ShapeDtypeStruct((S, VTOT), jnp.float32),
        scratch_shapes=[
            pltpu.VMEM((2, S, VT), jnp.float32),
            pltpu.VMEM((S, LAST_W), jnp.float32),
            pltpu.SemaphoreType.DMA((2,)),
            pltpu.SemaphoreType.DMA,
        ],
        compiler_params=pltpu.CompilerParams(
            dimension_semantics=("arbitrary",),
            vmem_limit_bytes=100 * 1024 * 1024,
        ),
    )(hb, h1, h2, lt, e0b, e1p, e2p)

    return out.reshape(1, S, VTOT)


# speedup vs baseline: 1.0422x; 1.0422x over previous
import jax
import jax.numpy as jnp
from jax.experimental import pallas as pl
from jax.experimental.pallas import tpu as pltpu

S = 2048
DIM = 1024
V0, V1, V2 = 10000, 20000, 30000
VTOT = V0 + V1 + V2
K1, K2 = DIM // 4, DIM // 16
VT = 1024
NB = 59
LAST_W = VTOT - (NB - 1) * VT
T1_START = V0 // VT
T2_START = (V0 + V1) // VT
B1 = V0 - T1_START * VT
B2 = (V0 + V1) - T2_START * VT
SH1 = VT - B1
SH2 = VT - B2
NE1 = (V1 + VT - 1) // VT
NE2 = (V2 + VT - 1) // VT
NSLOT = 3
DN = (((1,), (1,)), ((), ()))


def _body(h_ref, p1_ref, p2_ref, cw_ref, cb_ref, e0_ref, e1a_ref, e1b_ref,
          e2a_ref, e2b_ref, out_ref, h1s, h2s, lts, scratch, last_scratch,
          sems, last_sem):
    v = pl.program_id(0)
    slot = jax.lax.rem(v, NSLOT)

    @pl.when(v == 0)
    def _prep():
        h = h_ref[...]
        h1s[...] = jax.lax.dot_general(
            h, p1_ref[...], (((1,), (0,)), ((), ())),
            preferred_element_type=jnp.float32).astype(jnp.bfloat16)
        h2s[...] = jax.lax.dot_general(
            h, p2_ref[...], (((1,), (0,)), ((), ())),
            preferred_element_type=jnp.float32).astype(jnp.bfloat16)
        lt = jax.lax.dot_general(h, cw_ref[...], DN,
                                 preferred_element_type=jnp.float32)
        lts[...] = lt + cb_ref[...]

    def out_copy(step, slot):
        return pltpu.make_async_copy(
            scratch.at[slot],
            out_ref.at[:, pl.ds(step * VT, VT)],
            sems.at[slot])

    @pl.when(jnp.logical_and(v >= NSLOT, v < NB - 1))
    def _():
        out_copy(v - NSLOT, slot).wait()

    def head_mm():
        e0 = e0_ref[...].astype(jnp.bfloat16)
        return jax.lax.dot_general(h_ref[...].astype(jnp.bfloat16), e0, DN,
                                   preferred_element_type=jnp.float32)

    def t1_mm():
        e1 = jnp.concatenate([e1a_ref[SH1:, :], e1b_ref[:SH1, :]],
                             axis=0).astype(jnp.bfloat16)
        mm = jax.lax.dot_general(h1s[...], e1, DN,
                                 preferred_element_type=jnp.float32)
        return mm + lts[:, 0:1]

    def t2_mm():
        e2 = jnp.concatenate([e2a_ref[SH2:, :], e2b_ref[:SH2, :]],
                             axis=0).astype(jnp.bfloat16)
        mm = jax.lax.dot_general(h2s[...], e2, DN,
                                 preferred_element_type=jnp.float32)
        return mm + lts[:, 1:2]

    col = jax.lax.broadcasted_iota(jnp.int32, (1, VT), 1)

    @pl.when(v < T1_START)
    def _head():
        scratch[slot] = head_mm()

    @pl.when(v == T1_START)
    def _mixed1():
        scratch[slot] = jnp.where(col < B1, head_mm(), t1_mm())

    @pl.when(jnp.logical_and(v > T1_START, v < T2_START))
    def _tail1():
        scratch[slot] = t1_mm()

    @pl.when(v == T2_START)
    def _mixed2():
        scratch[slot] = jnp.where(col < B2, t1_mm(), t2_mm())

    @pl.when(jnp.logical_and(v > T2_START, v < NB - 1))
    def _tail2():
        scratch[slot] = t2_mm()

    @pl.when(v < NB - 1)
    def _():
        out_copy(v, slot).start()

    last_copy = pltpu.make_async_copy(
        last_scratch,
        out_ref.at[:, pl.ds((NB - 1) * VT, LAST_W)],
        last_sem)

    @pl.when(v == NB - 1)
    def _last():
        last_scratch[...] = t2_mm()[:, :LAST_W]
        last_copy.start()
        for i in range(1, NSLOT + 1):
            out_copy(NB - 1 - i, (NB - 1 - i) % NSLOT).wait()
        last_copy.wait()


def kernel(hidden, input, embed0_w, embed1_w, embed2_w, proj1_w, proj2_w,
           classes_w, classes_b):
    del input
    h = hidden.reshape(S, DIM)
    cb = classes_b.reshape(1, 2)

    out = pl.pallas_call(
        _body,
        grid=(NB,),
        in_specs=[
            pl.BlockSpec((S, DIM), lambda v: (0, 0)),
            pl.BlockSpec((DIM, K1), lambda v: (0, 0)),
            pl.BlockSpec((DIM, K2), lambda v: (0, 0)),
            pl.BlockSpec((2, DIM), lambda v: (0, 0)),
            pl.BlockSpec((1, 2), lambda v: (0, 0)),
            pl.BlockSpec((VT, DIM), lambda v: (jnp.minimum(v, T1_START), 0)),
            pl.BlockSpec((VT, K1),
                         lambda v: (jnp.clip(v - T1_START - 1, 0, NE1 - 1), 0)),
            pl.BlockSpec((VT, K1),
                         lambda v: (jnp.clip(v - T1_START, 0, NE1 - 1), 0)),
            pl.BlockSpec((VT, K2),
                         lambda v: (jnp.clip(v - T2_START - 1, 0, NE2 - 1), 0)),
            pl.BlockSpec((VT, K2),
                         lambda v: (jnp.clip(v - T2_START, 0, NE2 - 1), 0)),
        ],
        out_specs=pl.BlockSpec(memory_space=pl.ANY),
        out_shape=jax.ShapeDtypeStruct((S, VTOT), jnp.float32),
        scratch_shapes=[
            pltpu.VMEM((S, K1), jnp.bfloat16),
            pltpu.VMEM((S, K2), jnp.bfloat16),
            pltpu.VMEM((S, 2), jnp.float32),
            pltpu.VMEM((NSLOT, S, VT), jnp.float32),
            pltpu.VMEM((S, LAST_W), jnp.float32),
            pltpu.SemaphoreType.DMA((NSLOT,)),
            pltpu.SemaphoreType.DMA,
        ],
        compiler_params=pltpu.CompilerParams(
            dimension_semantics=("arbitrary",),
            vmem_limit_bytes=100 * 1024 * 1024,
        ),
    )(h, proj1_w, proj2_w, classes_w, cb, embed0_w, embed1_w, embed1_w,
      embed2_w, embed2_w)

    return out.reshape(1, S, VTOT)
